# Initial kernel scaffold; baseline (speedup 1.0000x reference)
import functools

import jax
import jax.numpy as jnp
from jax import lax
from jax.experimental import pallas as pl
from jax.experimental.pallas import tpu as pltpu
from jax.experimental.pallas import tpu_sc as plsc

N = 10000
E = 320000
D = 128

NC = 2
NS = 16
CHUNK = 128
NCHUNK = 80
E_PAD = NC * NS * NCHUNK * CHUNK
N_PAD = 10240
ROWS_PER_TILE = N_PAD // NS

_MESH = plsc.VectorSubcoreMesh(core_axis_name="c", subcore_axis_name="s")


@functools.partial(
    pl.kernel,
    out_type=jax.ShapeDtypeStruct((NC, N_PAD, 1), jnp.float32),
    mesh=_MESH,
    scratch_types=[
        pltpu.VMEM((NCHUNK, CHUNK), jnp.int32),
        pltpu.VMEM((CHUNK, 1), jnp.float32),
        pltpu.VMEM((ROWS_PER_TILE, 1), jnp.float32),
        pltpu.VMEM_SHARED((N_PAD, 1), jnp.float32),
    ],
)
def _deg_kernel(dst_hbm, ones_hbm, zeros_hbm, out_hbm, dst_vm, ones_vm, zvm,
                deg_sh):
    c = lax.axis_index("c")
    s = lax.axis_index("s")
    base = s * ROWS_PER_TILE

    pltpu.sync_copy(dst_hbm.at[c, s], dst_vm)
    pltpu.sync_copy(ones_hbm, ones_vm)
    pltpu.sync_copy(zeros_hbm, zvm)
    pltpu.sync_copy(zvm, deg_sh.at[pl.ds(base, ROWS_PER_TILE)])
    plsc.subcore_barrier()

    def body(j, carry):
        pltpu.sync_copy(ones_vm, deg_sh.at[dst_vm.at[j]], add=True)
        return carry

    lax.fori_loop(0, NCHUNK, body, 0)
    plsc.subcore_barrier()

    pltpu.sync_copy(deg_sh.at[pl.ds(base, ROWS_PER_TILE)], zvm)
    pltpu.sync_copy(zvm, out_hbm.at[c, pl.ds(base, ROWS_PER_TILE)])


@functools.partial(
    pl.kernel,
    out_type=jax.ShapeDtypeStruct((NC, N_PAD, D), jnp.float32),
    mesh=_MESH,
    scratch_types=[
        pltpu.VMEM((NCHUNK, CHUNK), jnp.int32),
        pltpu.VMEM((NCHUNK, CHUNK), jnp.int32),
        pltpu.VMEM((2, CHUNK, D), jnp.float32),
        pltpu.VMEM_SHARED((N_PAD, D), jnp.float32),
        pltpu.SemaphoreType.DMA,
        pltpu.SemaphoreType.DMA,
    ],
)
def _agg_kernel(g_hbm, src_hbm, dst_hbm, zeros_hbm, out_hbm, src_vm, dst_vm,
                rbuf, acc_sh, sem0, sem1):
    c = lax.axis_index("c")
    s = lax.axis_index("s")
    base = s * ROWS_PER_TILE
    sems = (sem0, sem1)

    pltpu.sync_copy(src_hbm.at[c, s], src_vm)
    pltpu.sync_copy(dst_hbm.at[c, s], dst_vm)

    pltpu.sync_copy(zeros_hbm, rbuf.at[0])
    for k in range(ROWS_PER_TILE // CHUNK):
        pltpu.sync_copy(rbuf.at[0], acc_sh.at[pl.ds(base + k * CHUNK, CHUNK)])
    plsc.subcore_barrier()

    pltpu.async_copy(g_hbm.at[src_vm.at[0]], rbuf.at[0], sem0)
    pltpu.async_copy(g_hbm.at[src_vm.at[1]], rbuf.at[1], sem1)

    def outer(jo, carry):
        for b in range(2):
            j = jo * 2 + b
            pltpu.make_async_copy(g_hbm.at[src_vm.at[j]], rbuf.at[b],
                                  sems[b]).wait()
            pltpu.sync_copy(rbuf.at[b], acc_sh.at[dst_vm.at[j]], add=True)
            nxt = j + 2

            @pl.when(nxt < NCHUNK)
            def _():
                pltpu.async_copy(g_hbm.at[src_vm.at[nxt]], rbuf.at[b], sems[b])

        return carry

    lax.fori_loop(0, NCHUNK // 2, outer, 0)
    plsc.subcore_barrier()

    for k in range(ROWS_PER_TILE // CHUNK):
        pltpu.sync_copy(acc_sh.at[pl.ds(base + k * CHUNK, CHUNK)], rbuf.at[0])
        pltpu.sync_copy(rbuf.at[0], out_hbm.at[c, pl.ds(base + k * CHUNK, CHUNK)])


_BLK = 2000


def _tc1_body(dega_ref, degb_ref, x_ref, w1_ref, g_ref, dinv_ref):
    deg = dega_ref[...] + degb_ref[...] + 1.0
    dinv = lax.rsqrt(deg)
    h = jnp.dot(x_ref[...], w1_ref[...], preferred_element_type=jnp.float32)
    g_ref[...] = h * dinv
    dinv_ref[...] = dinv


def _tc2_body(agga_ref, aggb_ref, g_ref, dinv_ref, b1_ref, w2_ref, g2_ref):
    dinv = dinv_ref[...]
    pre = dinv * (agga_ref[...] + aggb_ref[...] + g_ref[...]) + b1_ref[...]
    h1 = jnp.maximum(pre, 0.0)
    h2 = jnp.dot(h1, w2_ref[...], preferred_element_type=jnp.float32)
    g2_ref[...] = h2 * dinv


def _tc3_body(agga_ref, aggb_ref, g_ref, dinv_ref, b2_ref, out_ref):
    pre = dinv_ref[...] * (agga_ref[...] + aggb_ref[...] + g_ref[...]) + b2_ref[...]
    out_ref[...] = jnp.maximum(pre, 0.0)


def _row_spec(width):
    return pl.BlockSpec((_BLK, width), lambda i: (i, 0))


def _full_spec(shape):
    return pl.BlockSpec(shape, lambda i: (0, 0))


def _tc1(dega, degb, x, W1):
    return pl.pallas_call(
        _tc1_body,
        grid=(N // _BLK,),
        in_specs=[_row_spec(1), _row_spec(1), _row_spec(D), _full_spec((D, D))],
        out_specs=[_row_spec(D), _row_spec(1)],
        out_shape=[
            jax.ShapeDtypeStruct((N, D), jnp.float32),
            jax.ShapeDtypeStruct((N, 1), jnp.float32),
        ],
    )(dega, degb, x, W1)


def _tc2(agga, aggb, g, dinv, b1, W2):
    return pl.pallas_call(
        _tc2_body,
        grid=(N // _BLK,),
        in_specs=[_row_spec(D), _row_spec(D), _row_spec(D), _row_spec(1),
                  _full_spec((1, D)), _full_spec((D, D))],
        out_specs=_row_spec(D),
        out_shape=jax.ShapeDtypeStruct((N, D), jnp.float32),
    )(agga, aggb, g, dinv, b1, W2)


def _tc3(agga, aggb, g, dinv, b2):
    return pl.pallas_call(
        _tc3_body,
        grid=(N // _BLK,),
        in_specs=[_row_spec(D), _row_spec(D), _row_spec(D), _row_spec(1),
                  _full_spec((1, D))],
        out_specs=_row_spec(D),
        out_shape=jax.ShapeDtypeStruct((N, D), jnp.float32),
    )(agga, aggb, g, dinv, b2)


def kernel(x, edge_index, W1, b1, W2, b2):
    src = edge_index[0].astype(jnp.int32)
    dst = edge_index[1].astype(jnp.int32)

    pad = E_PAD - E
    pad_ar = jnp.arange(pad, dtype=jnp.int32)
    src_p = jnp.concatenate([src, pad_ar % N]).reshape(NC, NS, NCHUNK, CHUNK)
    dst_p = jnp.concatenate([dst, N + pad_ar % (N_PAD - N)]).reshape(
        NC, NS, NCHUNK, CHUNK)

    ones_col = jnp.ones((CHUNK, 1), jnp.float32)
    zeros_col = jnp.zeros((ROWS_PER_TILE, 1), jnp.float32)
    zeros_rows = jnp.zeros((CHUNK, D), jnp.float32)

    deg_parts = _deg_kernel(dst_p, ones_col, zeros_col)
    dega = deg_parts[0, :N]
    degb = deg_parts[1, :N]

    g1, dinv = _tc1(dega, degb, x, W1)

    agg1 = _agg_kernel(g1, src_p, dst_p, zeros_rows)
    g2 = _tc2(agg1[0, :N], agg1[1, :N], g1, dinv, b1.reshape(1, D), W2)

    agg2 = _agg_kernel(g2, src_p, dst_p, zeros_rows)
    out = _tc3(agg2[0, :N], agg2[1, :N], g2, dinv, b2.reshape(1, D))
    return out


# speedup vs baseline: 24.5777x; 24.5777x over previous
import functools

import jax
import jax.numpy as jnp
from jax import lax
from jax.experimental import pallas as pl
from jax.experimental.pallas import tpu as pltpu
from jax.experimental.pallas import tpu_sc as plsc

N = 10000
E = 320000
D = 128

NC = 2
NS = 16
CHUNK = 128
NCHUNK = 80
E_PAD = NC * NS * NCHUNK * CHUNK
N_PAD = 10240
ROWS_PER_TILE = N_PAD // NS

_MESH = plsc.VectorSubcoreMesh(core_axis_name="c", subcore_axis_name="s",
                               num_cores=NC, num_subcores=NS)


GCH_D = 16
DEGW = D


@functools.partial(
    pl.kernel,
    out_type=jax.ShapeDtypeStruct((NC, N_PAD, DEGW), jnp.float32),
    mesh=_MESH,
    scratch_types=[
        pltpu.VMEM((GCH_D, CHUNK), jnp.int32),
        pltpu.VMEM((CHUNK, DEGW), jnp.float32),
        pltpu.VMEM_SHARED((N_PAD, DEGW), jnp.float32),
    ],
)
def _deg_kernel(dst_hbm, ones_hbm, zeros_hbm, out_hbm, dst_vm, buf, deg_sh):
    c = lax.axis_index("c")
    s = lax.axis_index("s")
    base = s * ROWS_PER_TILE

    pltpu.sync_copy(zeros_hbm, buf)
    for k in range(ROWS_PER_TILE // CHUNK):
        pltpu.sync_copy(buf, deg_sh.at[pl.ds(base + k * CHUNK, CHUNK)])
    plsc.subcore_barrier()

    pltpu.sync_copy(ones_hbm, buf)
    for grp in range(NCHUNK // GCH_D):
        pltpu.sync_copy(dst_hbm.at[c, s, pl.ds(grp * GCH_D, GCH_D)], dst_vm)

        def body(j, carry):
            pltpu.sync_copy(buf, deg_sh.at[dst_vm.at[j]], add=True)
            return carry

        lax.fori_loop(0, GCH_D, body, 0)
    plsc.subcore_barrier()

    for k in range(ROWS_PER_TILE // CHUNK):
        pltpu.sync_copy(deg_sh.at[pl.ds(base + k * CHUNK, CHUNK)], buf)
        pltpu.sync_copy(buf, out_hbm.at[c, pl.ds(base + k * CHUNK, CHUNK)])


ACH = 128
ANCHUNK = E_PAD // (NC * NS * ACH)
NBUF = 2
GCH = 16
NGROUP = ANCHUNK // GCH


@functools.partial(
    pl.kernel,
    out_type=jax.ShapeDtypeStruct((NC, N_PAD, D), jnp.float32),
    mesh=_MESH,
    scratch_types=[
        pltpu.VMEM((GCH, ACH), jnp.int32),
        pltpu.VMEM((GCH, ACH), jnp.int32),
        pltpu.VMEM((NBUF, ACH, D), jnp.float32),
        pltpu.VMEM_SHARED((N_PAD, D), jnp.float32),
        [pltpu.SemaphoreType.DMA] * NBUF,
    ],
)
def _agg_kernel(g_hbm, src_hbm, dst_hbm, zeros_hbm, out_hbm, src_vm, dst_vm,
                rbuf, acc_sh, semg):
    c = lax.axis_index("c")
    s = lax.axis_index("s")
    base = s * ROWS_PER_TILE

    pltpu.sync_copy(zeros_hbm, rbuf.at[0])
    for k in range(ROWS_PER_TILE // ACH):
        pltpu.sync_copy(rbuf.at[0], acc_sh.at[pl.ds(base + k * ACH, ACH)])
    plsc.subcore_barrier()

    def gather(j, buf):
        pltpu.async_copy(g_hbm.at[src_vm.at[j]], rbuf.at[buf], semg[buf])

    def wait_gather(j, buf):
        pltpu.make_async_copy(g_hbm.at[src_vm.at[j]], rbuf.at[buf],
                              semg[buf]).wait()

    for grp in range(NGROUP):
        pltpu.sync_copy(src_hbm.at[c, s, pl.ds(grp * GCH, GCH)], src_vm)
        pltpu.sync_copy(dst_hbm.at[c, s, pl.ds(grp * GCH, GCH)], dst_vm)

        gather(0, 0)
        gather(1, 1)

        def outer(jo, carry):
            for b in range(NBUF):
                j = jo * NBUF + b
                wait_gather(j, b)
                pltpu.sync_copy(rbuf.at[b], acc_sh.at[dst_vm.at[j]], add=True)
                nxt = j + NBUF

                @pl.when(nxt < GCH)
                def _():
                    gather(nxt, b)

            return carry

        lax.fori_loop(0, GCH // NBUF, outer, 0)
    plsc.subcore_barrier()

    for k in range(ROWS_PER_TILE // ACH):
        pltpu.sync_copy(acc_sh.at[pl.ds(base + k * ACH, ACH)], rbuf.at[0])
        pltpu.sync_copy(rbuf.at[0], out_hbm.at[c, pl.ds(base + k * ACH, ACH)])


_BLK = 2000


def _tc1_body(dega_ref, degb_ref, x_ref, w1_ref, g_ref, dinv_ref):
    deg = dega_ref[...] + degb_ref[...] + 1.0
    dinv = lax.rsqrt(deg)
    h = jnp.dot(x_ref[...], w1_ref[...], preferred_element_type=jnp.float32)
    g_ref[...] = h * dinv
    dinv_ref[...] = dinv


def _tc2_body(agga_ref, aggb_ref, g_ref, dinv_ref, b1_ref, w2_ref, g2_ref):
    dinv = dinv_ref[...]
    pre = dinv * (agga_ref[...] + aggb_ref[...] + g_ref[...]) + b1_ref[...]
    h1 = jnp.maximum(pre, 0.0)
    h2 = jnp.dot(h1, w2_ref[...], preferred_element_type=jnp.float32)
    g2_ref[...] = h2 * dinv


def _tc3_body(agga_ref, aggb_ref, g_ref, dinv_ref, b2_ref, out_ref):
    pre = dinv_ref[...] * (agga_ref[...] + aggb_ref[...] + g_ref[...]) + b2_ref[...]
    out_ref[...] = jnp.maximum(pre, 0.0)


def _row_spec(width):
    return pl.BlockSpec((_BLK, width), lambda i: (i, 0))


def _full_spec(shape):
    return pl.BlockSpec(shape, lambda i: (0, 0))


def _tc1(dega, degb, x, W1):
    return pl.pallas_call(
        _tc1_body,
        grid=(N // _BLK,),
        in_specs=[_row_spec(1), _row_spec(1), _row_spec(D), _full_spec((D, D))],
        out_specs=[_row_spec(D), _row_spec(1)],
        out_shape=[
            jax.ShapeDtypeStruct((N, D), jnp.float32),
            jax.ShapeDtypeStruct((N, 1), jnp.float32),
        ],
    )(dega, degb, x, W1)


def _tc2(agga, aggb, g, dinv, b1, W2):
    return pl.pallas_call(
        _tc2_body,
        grid=(N // _BLK,),
        in_specs=[_row_spec(D), _row_spec(D), _row_spec(D), _row_spec(1),
                  _full_spec((1, D)), _full_spec((D, D))],
        out_specs=_row_spec(D),
        out_shape=jax.ShapeDtypeStruct((N, D), jnp.float32),
    )(agga, aggb, g, dinv, b1, W2)


def _tc3(agga, aggb, g, dinv, b2):
    return pl.pallas_call(
        _tc3_body,
        grid=(N // _BLK,),
        in_specs=[_row_spec(D), _row_spec(D), _row_spec(D), _row_spec(1),
                  _full_spec((1, D))],
        out_specs=_row_spec(D),
        out_shape=jax.ShapeDtypeStruct((N, D), jnp.float32),
    )(agga, aggb, g, dinv, b2)


def kernel(x, edge_index, W1, b1, W2, b2):
    src = edge_index[0].astype(jnp.int32)
    dst = edge_index[1].astype(jnp.int32)

    pad = E_PAD - E
    pad_ar = jnp.arange(pad, dtype=jnp.int32)
    src_flat = jnp.concatenate([src, pad_ar % N])
    dst_flat = jnp.concatenate([dst, N + pad_ar % (N_PAD - N)])
    src_p = src_flat.reshape(NC, NS, ANCHUNK, ACH)
    dst_p = dst_flat.reshape(NC, NS, ANCHUNK, ACH)
    dst_deg = dst_flat.reshape(NC, NS, NCHUNK, CHUNK)

    zeros_rows = jnp.zeros((ACH, D), jnp.float32)
    zeros_deg = jnp.zeros((CHUNK, DEGW), jnp.float32)
    ones_deg = jnp.ones((CHUNK, DEGW), jnp.float32)

    deg_parts = _deg_kernel(dst_deg, ones_deg, zeros_deg)
    dega = deg_parts[0, :N, :1]
    degb = deg_parts[1, :N, :1]

    g1, dinv = _tc1(dega, degb, x, W1)

    agg1 = _agg_kernel(g1, src_p, dst_p, zeros_rows)
    g2 = _tc2(agg1[0, :N], agg1[1, :N], g1, dinv, b1.reshape(1, D), W2)

    agg2 = _agg_kernel(g2, src_p, dst_p, zeros_rows)
    out = _tc3(agg2[0, :N], agg2[1, :N], g2, dinv, b2.reshape(1, D))
    return out
